# Initial kernel scaffold; baseline (speedup 1.0000x reference)
import jax
import jax.numpy as jnp
from jax.experimental import pallas as pl


def kernel(arg0_1, arg3_1, convert_element_type, convert_element_type_1):
    raise NotImplementedError("write your pallas kernel here")


# speedup vs baseline: 34.8077x; 34.8077x over previous
import functools

import jax
import jax.numpy as jnp
from jax import lax
from jax.experimental import pallas as pl
from jax.experimental.pallas import tpu as pltpu
from jax.experimental.pallas import tpu_sc as plsc

N_NODES = 10000
E = 320000
D = 128

CPT = 16
NGROUPS = D // CPT
NODE_H = N_NODES // 2
ACC = NODE_H * CPT
ROWS_PER_TILE = E // 2
R = 1000
NCHUNK = ROWS_PER_TILE // R

_mesh = plsc.VectorSubcoreMesh(core_axis_name="c", subcore_axis_name="s")
_sc_params = pltpu.CompilerParams(
    use_tc_tiling_on_sc=False, needs_layout_passes=False)


@functools.partial(
    pl.kernel,
    mesh=_mesh,
    out_type=jax.ShapeDtypeStruct((2, 2, NGROUPS, ACC), jnp.float32),
    scratch_types=[
        pltpu.VMEM((R, CPT), jnp.int32),
        pltpu.VMEM((R, CPT), jnp.float32),
        pltpu.VMEM((ACC,), jnp.float32),
        pltpu.SemaphoreType.DMA,
        pltpu.SemaphoreType.DMA,
    ],
    compiler_params=_sc_params,
)
def _scatter(idx_hbm, src_hbm, p_hbm, idx_v, src_v, acc_v, sem_i, sem_s):
    c = lax.axis_index("c")
    s = lax.axis_index("s")
    g = s % NGROUPS
    rh = s // NGROUPS
    nh = c
    col0 = g * CPT
    row0 = rh * ROWS_PER_TILE
    node_lo = nh * NODE_H

    lane = lax.iota(jnp.int32, 16)
    zeros = jnp.zeros((16,), jnp.float32)

    def zbody(i, carry):
        acc_v[pl.ds(i * 16, 16)] = zeros
        return carry

    lax.fori_loop(0, ACC // 16, zbody, 0)

    def chunk_body(k, carry):
        r0 = row0 + k * R
        cp_i = pltpu.async_copy(
            idx_hbm.at[pl.ds(r0, R), pl.ds(col0, CPT)], idx_v, sem_i)
        cp_s = pltpu.async_copy(
            src_hbm.at[pl.ds(r0, R), pl.ds(col0, CPT)], src_v, sem_s)
        cp_i.wait()
        cp_s.wait()

        def body(i, c2):
            iv = idx_v[i, :]
            sv = src_v[i, :]
            rel = iv - node_lo
            a = rel * CPT + lane
            m = (rel >= 0) & (rel < NODE_H)
            a = jnp.where(m, a, 0)
            plsc.addupdate_scatter(acc_v, [a], sv, mask=m)
            return c2

        lax.fori_loop(0, R, body, 0)
        return carry

    lax.fori_loop(0, NCHUNK, chunk_body, 0)

    pltpu.sync_copy(acc_v, p_hbm.at[rh, nh, g])


@functools.partial(
    pl.kernel,
    mesh=_mesh,
    out_type=jax.ShapeDtypeStruct((N_NODES, D), jnp.float32),
    scratch_types=[
        pltpu.VMEM((625, CPT), jnp.float32),
        pltpu.VMEM((625, CPT), jnp.float32),
        pltpu.VMEM((625 * CPT,), jnp.float32),
        pltpu.VMEM((625 * CPT,), jnp.float32),
        pltpu.VMEM((625, CPT), jnp.float32),
        pltpu.SemaphoreType.DMA,
    ],
    compiler_params=_sc_params,
)
def _combine(a3_hbm, base_hbm, p_hbm, out_hbm,
             a3_v, base_v, p0_v, p1_v, out_v, sem):
    c = lax.axis_index("c")
    s = lax.axis_index("s")
    g = s % NGROUPS
    q = (s // NGROUPS) * 2 + c
    col0 = g * CPT
    nq = N_NODES // 4
    node0 = q * nq
    nh = q // 2
    half_off = (q % 2) * (ACC // 2)
    CN = 625
    CW = CN * CPT

    def sub(t, carry):
        n0 = node0 + t * CN
        off = half_off + t * CW
        cps = [
            pltpu.async_copy(a3_hbm.at[pl.ds(n0, CN), pl.ds(col0, CPT)],
                             a3_v, sem),
            pltpu.async_copy(base_hbm.at[pl.ds(n0, CN), pl.ds(col0, CPT)],
                             base_v, sem),
            pltpu.async_copy(p_hbm.at[0, nh, g, pl.ds(off, CW)], p0_v, sem),
            pltpu.async_copy(p_hbm.at[1, nh, g, pl.ds(off, CW)], p1_v, sem),
        ]
        for cp in cps:
            cp.wait()

        def body(i, c2):
            tot = (base_v[i, :] + p0_v[pl.ds(i * 16, 16)]
                   + p1_v[pl.ds(i * 16, 16)])
            out_v[i, :] = a3_v[i, :] / tot
            return c2

        lax.fori_loop(0, CN, body, 0)

        pltpu.sync_copy(out_v, out_hbm.at[pl.ds(n0, CN), pl.ds(col0, CPT)])
        return carry

    lax.fori_loop(0, nq // CN, sub, 0)


@jax.jit
def _run(idx, a3, base, src):
    p = _scatter(idx, src)
    return _combine(a3, base, p)


def kernel(arg0_1, arg3_1, convert_element_type, convert_element_type_1):
    return (_run(arg0_1, arg3_1, convert_element_type, convert_element_type_1),)


# speedup vs baseline: 47.0263x; 1.3510x over previous
import functools

import jax
import jax.numpy as jnp
from jax import lax
from jax.experimental import pallas as pl
from jax.experimental.pallas import tpu as pltpu
from jax.experimental.pallas import tpu_sc as plsc

N_NODES = 10000
E = 320000
D = 128

CPT = 16
NGROUPS = D // CPT
NODE_H = N_NODES // 2
ACC = NODE_H * CPT
ROWS_PER_TILE = E // 2
R = 640
NCHUNK = ROWS_PER_TILE // R
UNROLL = 8

_mesh = plsc.VectorSubcoreMesh(core_axis_name="c", subcore_axis_name="s")
_sc_params = pltpu.CompilerParams(
    use_tc_tiling_on_sc=False, needs_layout_passes=False)


@functools.partial(
    pl.kernel,
    mesh=_mesh,
    out_type=jax.ShapeDtypeStruct((2, 2, NGROUPS, ACC), jnp.float32),
    scratch_types=[
        pltpu.VMEM((R, CPT), jnp.int32),
        pltpu.VMEM((R, CPT), jnp.int32),
        pltpu.VMEM((R, CPT), jnp.float32),
        pltpu.VMEM((R, CPT), jnp.float32),
        pltpu.VMEM((ACC,), jnp.float32),
        pltpu.SemaphoreType.DMA,
        pltpu.SemaphoreType.DMA,
        pltpu.SemaphoreType.DMA,
        pltpu.SemaphoreType.DMA,
    ],
    compiler_params=_sc_params,
)
def _scatter(idx_hbm, src_hbm, p_hbm,
             idx_v0, idx_v1, src_v0, src_v1, acc_v,
             sem_i0, sem_i1, sem_s0, sem_s1):
    c = lax.axis_index("c")
    s = lax.axis_index("s")
    g = s % NGROUPS
    rh = s // NGROUPS
    nh = c
    col0 = g * CPT
    row0 = rh * ROWS_PER_TILE
    node_lo = nh * NODE_H

    lane = lax.iota(jnp.int32, 16)
    lane_off = lane - node_lo * CPT
    zeros = jnp.zeros((16,), jnp.float32)

    def zbody(i, carry):
        acc_v[pl.ds(i * 16, 16)] = zeros
        return carry

    lax.fori_loop(0, ACC // 16, zbody, 0)

    bufs = ((idx_v0, src_v0, sem_i0, sem_s0),
            (idx_v1, src_v1, sem_i1, sem_s1))

    def _start(k, idx_b, src_b, sem_i, sem_s):
        r0 = row0 + k * R
        pltpu.async_copy(
            idx_hbm.at[pl.ds(r0, R), pl.ds(col0, CPT)], idx_b, sem_i)
        pltpu.async_copy(
            src_hbm.at[pl.ds(r0, R), pl.ds(col0, CPT)], src_b, sem_s)

    def _wait(idx_b, src_b, sem_i, sem_s):
        pltpu.make_async_copy(
            idx_hbm.at[pl.ds(0, R), pl.ds(col0, CPT)], idx_b, sem_i).wait()
        pltpu.make_async_copy(
            src_hbm.at[pl.ds(0, R), pl.ds(col0, CPT)], src_b, sem_s).wait()

    _start(0, *bufs[0])
    _start(1, *bufs[1])

    def chunk_pair(k2, carry):
        for b in range(2):
            k = 2 * k2 + b
            idx_b, src_b, sem_i, sem_s = bufs[b]
            _wait(idx_b, src_b, sem_i, sem_s)

            def body(j, c2):
                i0 = j * UNROLL
                for u in range(UNROLL):
                    i = i0 + u
                    iv = idx_b[i, :]
                    sv = src_b[i, :]
                    a = iv * CPT + lane_off
                    m = a.astype(jnp.uint32) < jnp.uint32(ACC)
                    a = jnp.where(m, a, 0)
                    plsc.addupdate_scatter(acc_v, [a], sv, mask=m)
                return c2

            lax.fori_loop(0, R // UNROLL, body, 0)

            @pl.when(k + 2 < NCHUNK)
            def _():
                _start(k + 2, idx_b, src_b, sem_i, sem_s)
        return carry

    lax.fori_loop(0, NCHUNK // 2, chunk_pair, 0)

    pltpu.sync_copy(acc_v, p_hbm.at[rh, nh, g])


@functools.partial(
    pl.kernel,
    mesh=_mesh,
    out_type=jax.ShapeDtypeStruct((N_NODES, D), jnp.float32),
    scratch_types=[
        pltpu.VMEM((625, CPT), jnp.float32),
        pltpu.VMEM((625, CPT), jnp.float32),
        pltpu.VMEM((625 * CPT,), jnp.float32),
        pltpu.VMEM((625 * CPT,), jnp.float32),
        pltpu.VMEM((625, CPT), jnp.float32),
        pltpu.SemaphoreType.DMA,
    ],
    compiler_params=_sc_params,
)
def _combine(a3_hbm, base_hbm, p_hbm, out_hbm,
             a3_v, base_v, p0_v, p1_v, out_v, sem):
    c = lax.axis_index("c")
    s = lax.axis_index("s")
    g = s % NGROUPS
    q = (s // NGROUPS) * 2 + c
    col0 = g * CPT
    nq = N_NODES // 4
    node0 = q * nq
    nh = q // 2
    half_off = (q % 2) * (ACC // 2)
    CN = 625
    CW = CN * CPT

    def sub(t, carry):
        n0 = node0 + t * CN
        off = half_off + t * CW
        cps = [
            pltpu.async_copy(a3_hbm.at[pl.ds(n0, CN), pl.ds(col0, CPT)],
                             a3_v, sem),
            pltpu.async_copy(base_hbm.at[pl.ds(n0, CN), pl.ds(col0, CPT)],
                             base_v, sem),
            pltpu.async_copy(p_hbm.at[0, nh, g, pl.ds(off, CW)], p0_v, sem),
            pltpu.async_copy(p_hbm.at[1, nh, g, pl.ds(off, CW)], p1_v, sem),
        ]
        for cp in cps:
            cp.wait()

        def body(i, c2):
            tot = (base_v[i, :] + p0_v[pl.ds(i * 16, 16)]
                   + p1_v[pl.ds(i * 16, 16)])
            out_v[i, :] = a3_v[i, :] / tot
            return c2

        lax.fori_loop(0, CN, body, 0)

        pltpu.sync_copy(out_v, out_hbm.at[pl.ds(n0, CN), pl.ds(col0, CPT)])
        return carry

    lax.fori_loop(0, nq // CN, sub, 0)


@jax.jit
def _run(idx, a3, base, src):
    p = _scatter(idx, src)
    return _combine(a3, base, p)


def kernel(arg0_1, arg3_1, convert_element_type, convert_element_type_1):
    return (_run(arg0_1, arg3_1, convert_element_type, convert_element_type_1),)


# speedup vs baseline: 147.7889x; 3.1427x over previous
import functools

import jax
import jax.numpy as jnp
from jax import lax
from jax.experimental import pallas as pl
from jax.experimental.pallas import tpu as pltpu
from jax.experimental.pallas import tpu_sc as plsc

N_NODES = 10000
E = 320000
D = 128

CPT = 16
NGROUPS = D // CPT
NODE_H = N_NODES // 2
ACC = NODE_H * CPT
ROWS_PER_TILE = E // 2
R = 640
NCHUNK = ROWS_PER_TILE // R
UNROLL = 8

_mesh = plsc.VectorSubcoreMesh(core_axis_name="c", subcore_axis_name="s")
_sc_params = pltpu.CompilerParams(
    use_tc_tiling_on_sc=False, needs_layout_passes=False)


@functools.partial(
    pl.kernel,
    mesh=_mesh,
    out_type=jax.ShapeDtypeStruct((2, 2, NGROUPS, ACC), jnp.float32),
    scratch_types=[
        pltpu.VMEM((R, CPT), jnp.int32),
        pltpu.VMEM((R, CPT), jnp.int32),
        pltpu.VMEM((R, CPT), jnp.float32),
        pltpu.VMEM((R, CPT), jnp.float32),
        pltpu.VMEM((ACC,), jnp.float32),
        pltpu.SemaphoreType.DMA,
        pltpu.SemaphoreType.DMA,
        pltpu.SemaphoreType.DMA,
        pltpu.SemaphoreType.DMA,
    ],
    compiler_params=_sc_params,
)
def _scatter(idx_hbm, src_hbm, p_hbm,
             idx_v0, idx_v1, src_v0, src_v1, acc_v,
             sem_i0, sem_i1, sem_s0, sem_s1):
    c = lax.axis_index("c")
    s = lax.axis_index("s")
    g = s % NGROUPS
    rh = s // NGROUPS
    nh = c
    col0 = g * CPT
    row0 = rh * ROWS_PER_TILE
    node_lo = nh * NODE_H

    lane = lax.iota(jnp.int32, 16)
    lane_off = lane - node_lo * CPT
    zeros = jnp.zeros((16,), jnp.float32)

    def zbody(i, carry):
        acc_v[pl.ds(i * 16, 16)] = zeros
        return carry

    lax.fori_loop(0, ACC // 16, zbody, 0)

    bufs = ((idx_v0, src_v0, sem_i0, sem_s0),
            (idx_v1, src_v1, sem_i1, sem_s1))

    def _start(k, idx_b, src_b, sem_i, sem_s):
        r0 = row0 + k * R
        pltpu.async_copy(
            idx_hbm.at[pl.ds(r0, R), pl.ds(col0, CPT)], idx_b, sem_i)
        pltpu.async_copy(
            src_hbm.at[pl.ds(r0, R), pl.ds(col0, CPT)], src_b, sem_s)

    def _wait(idx_b, src_b, sem_i, sem_s):
        pltpu.make_async_copy(
            idx_hbm.at[pl.ds(0, R), pl.ds(col0, CPT)], idx_b, sem_i).wait()
        pltpu.make_async_copy(
            src_hbm.at[pl.ds(0, R), pl.ds(col0, CPT)], src_b, sem_s).wait()

    _start(0, *bufs[0])
    _start(1, *bufs[1])

    def chunk_pair(k2, carry):
        for b in range(2):
            k = 2 * k2 + b
            idx_b, src_b, sem_i, sem_s = bufs[b]
            _wait(idx_b, src_b, sem_i, sem_s)

            @plsc.parallel_loop(0, R, step=1, unroll=UNROLL)
            def _body(i):
                iv = idx_b[i, :]
                sv = src_b[i, :]
                a = iv * CPT + lane_off
                m = a.astype(jnp.uint32) < jnp.uint32(ACC)
                a = jnp.where(m, a, 0)
                plsc.addupdate_scatter(acc_v, [a], sv, mask=m)

            @pl.when(k + 2 < NCHUNK)
            def _():
                _start(k + 2, idx_b, src_b, sem_i, sem_s)
        return carry

    lax.fori_loop(0, NCHUNK // 2, chunk_pair, 0)

    pltpu.sync_copy(acc_v, p_hbm.at[rh, nh, g])


@functools.partial(
    pl.kernel,
    mesh=_mesh,
    out_type=jax.ShapeDtypeStruct((N_NODES, D), jnp.float32),
    scratch_types=[
        pltpu.VMEM((625, CPT), jnp.float32),
        pltpu.VMEM((625, CPT), jnp.float32),
        pltpu.VMEM((625 * CPT,), jnp.float32),
        pltpu.VMEM((625 * CPT,), jnp.float32),
        pltpu.VMEM((625, CPT), jnp.float32),
        pltpu.SemaphoreType.DMA,
    ],
    compiler_params=_sc_params,
)
def _combine(a3_hbm, base_hbm, p_hbm, out_hbm,
             a3_v, base_v, p0_v, p1_v, out_v, sem):
    c = lax.axis_index("c")
    s = lax.axis_index("s")
    g = s % NGROUPS
    q = (s // NGROUPS) * 2 + c
    col0 = g * CPT
    nq = N_NODES // 4
    node0 = q * nq
    nh = q // 2
    half_off = (q % 2) * (ACC // 2)
    CN = 625
    CW = CN * CPT

    def sub(t, carry):
        n0 = node0 + t * CN
        off = half_off + t * CW
        cps = [
            pltpu.async_copy(a3_hbm.at[pl.ds(n0, CN), pl.ds(col0, CPT)],
                             a3_v, sem),
            pltpu.async_copy(base_hbm.at[pl.ds(n0, CN), pl.ds(col0, CPT)],
                             base_v, sem),
            pltpu.async_copy(p_hbm.at[0, nh, g, pl.ds(off, CW)], p0_v, sem),
            pltpu.async_copy(p_hbm.at[1, nh, g, pl.ds(off, CW)], p1_v, sem),
        ]
        for cp in cps:
            cp.wait()

        def body(i, c2):
            tot = (base_v[i, :] + p0_v[pl.ds(i * 16, 16)]
                   + p1_v[pl.ds(i * 16, 16)])
            out_v[i, :] = a3_v[i, :] / tot
            return c2

        lax.fori_loop(0, CN, body, 0)

        pltpu.sync_copy(out_v, out_hbm.at[pl.ds(n0, CN), pl.ds(col0, CPT)])
        return carry

    lax.fori_loop(0, nq // CN, sub, 0)


@jax.jit
def _run(idx, a3, base, src):
    p = _scatter(idx, src)
    return _combine(a3, base, p)


def kernel(arg0_1, arg3_1, convert_element_type, convert_element_type_1):
    return (_run(arg0_1, arg3_1, convert_element_type, convert_element_type_1),)


# speedup vs baseline: 150.9731x; 1.0215x over previous
import functools

import jax
import jax.numpy as jnp
from jax import lax
from jax.experimental import pallas as pl
from jax.experimental.pallas import tpu as pltpu
from jax.experimental.pallas import tpu_sc as plsc

N_NODES = 10000
E = 320000
D = 128

CPT = 16
NGROUPS = D // CPT
NODE_H = N_NODES // 2
ACC = NODE_H * CPT
ROWS_PER_TILE = E // 2
R = 500
NCHUNK = ROWS_PER_TILE // R
UNROLL = 8

_mesh = plsc.VectorSubcoreMesh(core_axis_name="c", subcore_axis_name="s")
_sc_params = pltpu.CompilerParams(
    use_tc_tiling_on_sc=False, needs_layout_passes=False)


@functools.partial(
    pl.kernel,
    mesh=_mesh,
    out_type=jax.ShapeDtypeStruct((2, 2, NGROUPS, ACC), jnp.float32),
    scratch_types=[
        pltpu.VMEM((R, CPT), jnp.int32),
        pltpu.VMEM((R, CPT), jnp.int32),
        pltpu.VMEM((R, CPT), jnp.float32),
        pltpu.VMEM((R, CPT), jnp.float32),
        pltpu.VMEM((ACC,), jnp.float32),
        pltpu.VMEM_SHARED((R, D), jnp.int32),
        pltpu.VMEM_SHARED((R, D), jnp.int32),
        pltpu.VMEM_SHARED((R, D), jnp.float32),
        pltpu.VMEM_SHARED((R, D), jnp.float32),
        pltpu.SemaphoreType.DMA,
        pltpu.SemaphoreType.DMA,
        pltpu.SemaphoreType.DMA,
        pltpu.SemaphoreType.DMA,
    ],
    compiler_params=_sc_params,
)
def _scatter(idx_hbm, src_hbm, p_hbm,
             idx_v0, idx_v1, src_v0, src_v1, acc_v,
             st_i0, st_i1, st_s0, st_s1,
             sem_st0, sem_st1, sem_p0, sem_p1):
    c = lax.axis_index("c")
    s = lax.axis_index("s")
    g = s % NGROUPS
    nh = s // NGROUPS
    rh = c
    col0 = g * CPT
    row0 = rh * ROWS_PER_TILE
    node_lo = nh * NODE_H

    lane = lax.iota(jnp.int32, 16)
    lane_off = lane - node_lo * CPT
    zeros = jnp.zeros((16,), jnp.float32)

    def zbody(i, carry):
        acc_v[pl.ds(i * 16, 16)] = zeros
        return carry

    lax.fori_loop(0, ACC // 16, zbody, 0)

    stages = ((st_i0, st_s0, sem_st0), (st_i1, st_s1, sem_st1))
    tbufs = ((idx_v0, src_v0, sem_p0), (idx_v1, src_v1, sem_p1))

    def stage_start(k, b):
        r0 = row0 + k * R
        pltpu.async_copy(idx_hbm.at[pl.ds(r0, R), :], stages[b][0],
                         stages[b][2])
        pltpu.async_copy(src_hbm.at[pl.ds(r0, R), :], stages[b][1],
                         stages[b][2])

    def stage_wait(b):
        pltpu.make_async_copy(idx_hbm.at[pl.ds(0, R), :], stages[b][0],
                              stages[b][2]).wait()
        pltpu.make_async_copy(src_hbm.at[pl.ds(0, R), :], stages[b][1],
                              stages[b][2]).wait()

    def pull_start(b):
        pltpu.async_copy(stages[b][0].at[:, pl.ds(col0, CPT)], tbufs[b][0],
                         tbufs[b][2])
        pltpu.async_copy(stages[b][1].at[:, pl.ds(col0, CPT)], tbufs[b][1],
                         tbufs[b][2])

    def pull_wait(b):
        pltpu.make_async_copy(stages[b][0].at[:, pl.ds(col0, CPT)],
                              tbufs[b][0], tbufs[b][2]).wait()
        pltpu.make_async_copy(stages[b][1].at[:, pl.ds(col0, CPT)],
                              tbufs[b][1], tbufs[b][2]).wait()

    @pl.when(s == 0)
    def _():
        stage_start(0, 0)
        stage_start(1, 1)
        stage_wait(0)

    plsc.subcore_barrier()
    pull_start(0)

    def chunk_pair(k2, carry):
        for b in range(2):
            k = 2 * k2 + b
            idx_b, src_b, _ = tbufs[b]
            pull_wait(b)
            plsc.subcore_barrier()

            @pl.when(jnp.logical_and(s == 0, k + 2 < NCHUNK))
            def _():
                stage_start(k + 2, b)

            @pl.when(jnp.logical_and(s == 0, k + 1 < NCHUNK))
            def _():
                stage_wait(1 - b)

            plsc.subcore_barrier()

            @pl.when(k + 1 < NCHUNK)
            def _():
                pull_start(1 - b)

            @plsc.parallel_loop(0, R, step=1, unroll=UNROLL)
            def _body(i):
                iv = idx_b[i, :]
                sv = src_b[i, :]
                a = iv * CPT + lane_off
                m = a.astype(jnp.uint32) < jnp.uint32(ACC)
                a = jnp.where(m, a, 0)
                plsc.addupdate_scatter(acc_v, [a], sv, mask=m)

        return carry

    lax.fori_loop(0, NCHUNK // 2, chunk_pair, 0)

    pltpu.sync_copy(acc_v, p_hbm.at[rh, nh, g])


@functools.partial(
    pl.kernel,
    mesh=_mesh,
    out_type=jax.ShapeDtypeStruct((N_NODES, D), jnp.float32),
    scratch_types=[
        pltpu.VMEM((625, CPT), jnp.float32),
        pltpu.VMEM((625, CPT), jnp.float32),
        pltpu.VMEM((625 * CPT,), jnp.float32),
        pltpu.VMEM((625 * CPT,), jnp.float32),
        pltpu.VMEM((625, CPT), jnp.float32),
        pltpu.SemaphoreType.DMA,
    ],
    compiler_params=_sc_params,
)
def _combine(a3_hbm, base_hbm, p_hbm, out_hbm,
             a3_v, base_v, p0_v, p1_v, out_v, sem):
    c = lax.axis_index("c")
    s = lax.axis_index("s")
    g = s % NGROUPS
    q = (s // NGROUPS) * 2 + c
    col0 = g * CPT
    nq = N_NODES // 4
    node0 = q * nq
    nh = q // 2
    half_off = (q % 2) * (ACC // 2)
    CN = 625
    CW = CN * CPT

    def sub(t, carry):
        n0 = node0 + t * CN
        off = half_off + t * CW
        cps = [
            pltpu.async_copy(a3_hbm.at[pl.ds(n0, CN), pl.ds(col0, CPT)],
                             a3_v, sem),
            pltpu.async_copy(base_hbm.at[pl.ds(n0, CN), pl.ds(col0, CPT)],
                             base_v, sem),
            pltpu.async_copy(p_hbm.at[0, nh, g, pl.ds(off, CW)], p0_v, sem),
            pltpu.async_copy(p_hbm.at[1, nh, g, pl.ds(off, CW)], p1_v, sem),
        ]
        for cp in cps:
            cp.wait()

        def body(i, c2):
            tot = (base_v[i, :] + p0_v[pl.ds(i * 16, 16)]
                   + p1_v[pl.ds(i * 16, 16)])
            out_v[i, :] = a3_v[i, :] / tot
            return c2

        lax.fori_loop(0, CN, body, 0)

        pltpu.sync_copy(out_v, out_hbm.at[pl.ds(n0, CN), pl.ds(col0, CPT)])
        return carry

    lax.fori_loop(0, nq // CN, sub, 0)


@jax.jit
def _run(idx, a3, base, src):
    p = _scatter(idx, src)
    return _combine(a3, base, p)


def kernel(arg0_1, arg3_1, convert_element_type, convert_element_type_1):
    return (_run(arg0_1, arg3_1, convert_element_type, convert_element_type_1),)
